# baseline (speedup 1.0000x reference)
import functools

import jax
import jax.numpy as jnp
from jax import lax
from jax.experimental import pallas as pl
from jax.experimental.pallas import tpu as pltpu
from jax.experimental.pallas import tpu_sc as plsc

N = 200000
C = 64
SCALE = 4
GLOBAL_SIZE = 64
N_GLOBAL = GLOBAL_SIZE ** 3

NC = 2
NS = 16
NW = NC * NS

QB = 128
NQ = 3
B = QB * NQ
CNT = 18
PAIRS = CNT // 2


def _fusion_body(co_hbm, lf_hbm, gf_hbm, out_hbm,
                 coord0, coord1, idx0a, idx0b, idx0c, idx1a, idx1b, idx1c,
                 ga0, gb0, gc0, ga1, gb1, gc1, acc0, acc1,
                 csem0, csem1, gsem0, gsem1, lsem0, lsem1, ssem0, ssem1):
    wid = lax.axis_index("s") * NC + lax.axis_index("c")
    iota3 = lax.iota(jnp.int32, 16) * 3

    sets = (
        (coord0, (idx0a, idx0b, idx0c), (ga0, gb0, gc0), acc0,
         csem0, gsem0, lsem0, ssem0),
        (coord1, (idx1a, idx1b, idx1c), (ga1, gb1, gc1), acc1,
         csem1, gsem1, lsem1, ssem1),
    )

    def phase(j, c, s, s_next):
        coordv, idxs, gaths, acc, csem, _, lsem, ssem = sets[s]
        coordn, _, _, _, csemn, _, _, _ = sets[s_next]
        gsem = sets[s][5]
        base = jnp.minimum(c * B, N - B)

        pltpu.make_async_copy(
            co_hbm.at[pl.ds(3 * base, 3 * B)], coordv, csem).wait()

        for t in range(B // 16):
            p = iota3 + (t * 48)
            x = jnp.clip(plsc.load_gather(coordv, [p]) >> 2, 0, GLOBAL_SIZE - 1)
            y = jnp.clip(plsc.load_gather(coordv, [p + 1]) >> 2, 0, GLOBAL_SIZE - 1)
            z = jnp.clip(plsc.load_gather(coordv, [p + 2]) >> 2, 0, GLOBAL_SIZE - 1)
            flat = (x * (GLOBAL_SIZE * GLOBAL_SIZE) + y * GLOBAL_SIZE) + z
            idxs[t // (QB // 16)][pl.ds((t % (QB // 16)) * 16, 16)] = flat

        gcps = [pltpu.async_copy(gf_hbm.at[idxs[q]], gaths[q], gsem)
                for q in range(NQ)]

        @pl.when(j > 0)
        def _():
            pltpu.make_async_copy(acc, out_hbm.at[pl.ds(base, B)], ssem).wait()

        lcp = pltpu.async_copy(lf_hbm.at[pl.ds(base, B)], acc, lsem)

        nbase = jnp.minimum((c + NW) * B, N - B)
        pltpu.make_async_copy(
            co_hbm.at[pl.ds(3 * nbase, 3 * B)], coordn, csemn).start()

        for cp in gcps:
            cp.wait()
        lcp.wait()

        for q in range(NQ):
            gq = gaths[q]
            qbase = q * QB

            def add_rows(r, c2, gq=gq, qbase=qbase):
                r4 = r * 4
                for rr in range(4):
                    for cc in range(C // 16):
                        sl = pl.ds(cc * 16, 16)
                        acc[qbase + r4 + rr, sl] = (
                            acc[qbase + r4 + rr, sl] + gq[r4 + rr, sl])
                return c2

            lax.fori_loop(0, QB // 4, add_rows, 0)

        pltpu.make_async_copy(acc, out_hbm.at[pl.ds(base, B)], ssem).start()

    base0 = jnp.minimum(wid * B, N - B)
    pltpu.make_async_copy(
        co_hbm.at[pl.ds(3 * base0, 3 * B)], coord0, csem0).start()

    def pair_body(j, carry):
        c0 = wid + (2 * j) * NW
        phase(j, c0, 0, 1)
        phase(j, c0 + NW, 1, 0)
        return carry

    lax.fori_loop(0, PAIRS, pair_body, 0)

    tail = jnp.minimum((wid + CNT * NW) * B, N - B)
    pltpu.make_async_copy(
        co_hbm.at[pl.ds(3 * tail, 3 * B)], coord0, csem0).wait()
    pltpu.make_async_copy(acc0, out_hbm.at[pl.ds(N - B, B)], ssem0).wait()
    pltpu.make_async_copy(acc1, out_hbm.at[pl.ds(N - B, B)], ssem1).wait()


@jax.jit
def _fusion(co, lf, gf):
    mesh = plsc.VectorSubcoreMesh(core_axis_name="c", subcore_axis_name="s")
    return pl.kernel(
        _fusion_body,
        out_type=jax.ShapeDtypeStruct((N, C), jnp.float32),
        mesh=mesh,
        scratch_types=[
            pltpu.VMEM((3 * B,), jnp.int32),
            pltpu.VMEM((3 * B,), jnp.int32),
            pltpu.VMEM((QB,), jnp.int32),
            pltpu.VMEM((QB,), jnp.int32),
            pltpu.VMEM((QB,), jnp.int32),
            pltpu.VMEM((QB,), jnp.int32),
            pltpu.VMEM((QB,), jnp.int32),
            pltpu.VMEM((QB,), jnp.int32),
            pltpu.VMEM((QB, C), jnp.float32),
            pltpu.VMEM((QB, C), jnp.float32),
            pltpu.VMEM((QB, C), jnp.float32),
            pltpu.VMEM((QB, C), jnp.float32),
            pltpu.VMEM((QB, C), jnp.float32),
            pltpu.VMEM((QB, C), jnp.float32),
            pltpu.VMEM((B, C), jnp.float32),
            pltpu.VMEM((B, C), jnp.float32),
            pltpu.SemaphoreType.DMA,
            pltpu.SemaphoreType.DMA,
            pltpu.SemaphoreType.DMA,
            pltpu.SemaphoreType.DMA,
            pltpu.SemaphoreType.DMA,
            pltpu.SemaphoreType.DMA,
            pltpu.SemaphoreType.DMA,
            pltpu.SemaphoreType.DMA,
        ],
        compiler_params=pltpu.CompilerParams(
            use_tc_tiling_on_sc=False, needs_layout_passes=False),
    )(co, lf, gf)


def kernel(local_features, local_coords, local_base, global_features, global_base):
    adj = (local_coords.astype(jnp.int32)
           + local_base.astype(jnp.int32)[None, :]
           - SCALE * global_base.astype(jnp.int32)[None, :])
    co = adj.reshape(-1)
    return _fusion(co, local_features, global_features)


# speedup vs baseline: 1.1940x; 1.1940x over previous
import functools

import jax
import jax.numpy as jnp
from jax import lax
from jax.experimental import pallas as pl
from jax.experimental.pallas import tpu as pltpu
from jax.experimental.pallas import tpu_sc as plsc

N = 200000
C = 64
SCALE = 4
GLOBAL_SIZE = 64
N_GLOBAL = GLOBAL_SIZE ** 3

NC = 2
NS = 16
NW = NC * NS

QB = 128
B = 1024
G = (N + B - 1) // B
NP = G * B
H = B // 2


def _gather_body(cx_hbm, cy_hbm, cz_hbm, gf_hbm, g2_hbm,
                 cxv, cyv, czv, i0, i1, i2, i3, i4, i5, i6, i7,
                 d0, d1, d2, d3, d4, d5, d6, d7, gsem):
    wid = lax.axis_index("s") * NC + lax.axis_index("c")
    cnt = (G + NW - 1 - wid) // NW
    idxs = (i0, i1, i2, i3, i4, i5, i6, i7)
    dsts = (d0, d1, d2, d3, d4, d5, d6, d7)

    def chunk_body(j, carry):
        g = wid + j * NW
        base = g * B

        pltpu.sync_copy(cx_hbm.at[pl.ds(base, B)], cxv)
        pltpu.sync_copy(cy_hbm.at[pl.ds(base, B)], cyv)
        pltpu.sync_copy(cz_hbm.at[pl.ds(base, B)], czv)

        for t in range(B // 16):
            sl = pl.ds(t * 16, 16)
            x = jnp.clip(cxv[sl] >> 2, 0, GLOBAL_SIZE - 1)
            y = jnp.clip(cyv[sl] >> 2, 0, GLOBAL_SIZE - 1)
            z = jnp.clip(czv[sl] >> 2, 0, GLOBAL_SIZE - 1)
            flat = (x * (GLOBAL_SIZE * GLOBAL_SIZE) + y * GLOBAL_SIZE) + z
            idxs[t // 8][pl.ds((t % 8) * 16, 16)] = flat

        cps = [pltpu.async_copy(gf_hbm.at[idxs[q]], dsts[q], gsem)
               for q in range(8)]
        for cp in cps:
            cp.wait()

        base2 = g * H
        for q in range(4):
            pltpu.sync_copy(
                dsts[q], g2_hbm.at[pl.ds(base2 + q * QB, QB), pl.ds(0, C)])
            pltpu.sync_copy(
                dsts[4 + q], g2_hbm.at[pl.ds(base2 + q * QB, QB), pl.ds(C, C)])
        return carry

    lax.fori_loop(0, cnt, chunk_body, 0)


def _fuse_body(lf_ref, g2_ref, out_ref):
    lf = lf_ref[...]
    g2 = g2_ref[...]
    out_ref[0:H, :] = lf[0:H, :] + g2[:, 0:C]
    out_ref[H:B, :] = lf[H:B, :] + g2[:, C:2 * C]


@jax.jit
def _fusion(cx, cy, cz, lf, gf):
    mesh = plsc.VectorSubcoreMesh(core_axis_name="c", subcore_axis_name="s")
    g2 = pl.kernel(
        _gather_body,
        out_type=jax.ShapeDtypeStruct((G * H, 2 * C), jnp.float32),
        mesh=mesh,
        scratch_types=(
            [pltpu.VMEM((B,), jnp.int32)] * 3
            + [pltpu.VMEM((QB,), jnp.int32)] * 8
            + [pltpu.VMEM((QB, C), jnp.float32)] * 8
            + [pltpu.SemaphoreType.DMA]
        ),
        compiler_params=pltpu.CompilerParams(
            use_tc_tiling_on_sc=False, needs_layout_passes=False),
    )(cx, cy, cz, gf)

    out = pl.pallas_call(
        _fuse_body,
        grid=(G,),
        in_specs=[
            pl.BlockSpec((B, C), lambda i: (i, 0)),
            pl.BlockSpec((H, 2 * C), lambda i: (i, 0)),
        ],
        out_specs=pl.BlockSpec((B, C), lambda i: (i, 0)),
        out_shape=jax.ShapeDtypeStruct((N, C), jnp.float32),
    )(lf, g2)
    return out


def kernel(local_features, local_coords, local_base, global_features, global_base):
    adj = (local_coords.astype(jnp.int32)
           + local_base.astype(jnp.int32)[None, :]
           - SCALE * global_base.astype(jnp.int32)[None, :])
    pad = (0, NP - N)
    cx = jnp.pad(adj[:, 0], pad)
    cy = jnp.pad(adj[:, 1], pad)
    cz = jnp.pad(adj[:, 2], pad)
    return _fusion(cx, cy, cz, local_features, global_features)


# speedup vs baseline: 1.6337x; 1.3682x over previous
import functools

import jax
import jax.numpy as jnp
from jax import lax
from jax.experimental import pallas as pl
from jax.experimental.pallas import tpu as pltpu
from jax.experimental.pallas import tpu_sc as plsc

N = 200000
C = 64
SCALE = 4
GLOBAL_SIZE = 64
N_GLOBAL = GLOBAL_SIZE ** 3

NC = 2
NS = 16
NW = NC * NS

QB = 128
TB = 1024
H = TB // 2
NB = (N + TB - 1) // TB
NP = NB * TB
G = 2 * NB


def _gather_body(cx_hbm, cy_hbm, cz_hbm, gf_hbm, g2_hbm,
                 cx0, cy0, cz0, cx1, cy1, cz1,
                 i00, i01, i02, i03, i10, i11, i12, i13,
                 d00, d01, d02, d03, d10, d11, d12, d13,
                 csem0, csem1, gsem0, gsem1, ssem0, ssem1):
    wid = lax.axis_index("s") * NC + lax.axis_index("c")
    cnt = (G + NW - 1 - wid) // NW

    sets = (
        ((cx0, cy0, cz0), (i00, i01, i02, i03), (d00, d01, d02, d03),
         csem0, gsem0, ssem0),
        ((cx1, cy1, cz1), (i10, i11, i12, i13), (d10, d11, d12, d13),
         csem1, gsem1, ssem1),
    )

    def coords_of(c):
        return (c >> 1) * TB + (c & 1) * H

    def start_coords(c, s):
        (cxv, cyv, czv), _, _, csem, _, _ = sets[s]
        base = jnp.minimum(coords_of(c), NP - H)
        pltpu.make_async_copy(cx_hbm.at[pl.ds(base, H)], cxv, csem).start()
        pltpu.make_async_copy(cy_hbm.at[pl.ds(base, H)], cyv, csem).start()
        pltpu.make_async_copy(cz_hbm.at[pl.ds(base, H)], czv, csem).start()

    def phase(j, c, s):
        (cxv, cyv, czv), idxs, dsts, csem, gsem, ssem = sets[s]
        base = coords_of(c)
        half = c & 1
        base2 = (c >> 1) * H

        pltpu.make_async_copy(cx_hbm.at[pl.ds(base, H)], cxv, csem).wait()
        pltpu.make_async_copy(cy_hbm.at[pl.ds(base, H)], cyv, csem).wait()
        pltpu.make_async_copy(cz_hbm.at[pl.ds(base, H)], czv, csem).wait()

        for t in range(H // 16):
            sl = pl.ds(t * 16, 16)
            x = jnp.clip(cxv[sl] >> 2, 0, GLOBAL_SIZE - 1)
            y = jnp.clip(cyv[sl] >> 2, 0, GLOBAL_SIZE - 1)
            z = jnp.clip(czv[sl] >> 2, 0, GLOBAL_SIZE - 1)
            flat = (x * (GLOBAL_SIZE * GLOBAL_SIZE) + y * GLOBAL_SIZE) + z
            idxs[t // 8][pl.ds((t % 8) * 16, 16)] = flat

        gcps = [pltpu.async_copy(gf_hbm.at[idxs[q]], dsts[q], gsem)
                for q in range(4)]

        start_coords(c + NW, 1 - s)

        @pl.when(j > 1)
        def _():
            for q in range(4):
                pltpu.make_async_copy(
                    dsts[q],
                    g2_hbm.at[pl.ds(base2 + q * QB, QB),
                              pl.ds(half * C, C)],
                    ssem).wait()

        for cp in gcps:
            cp.wait()

        for q in range(4):
            pltpu.make_async_copy(
                dsts[q],
                g2_hbm.at[pl.ds(base2 + q * QB, QB), pl.ds(half * C, C)],
                ssem).start()

    start_coords(wid, 0)

    def pair_body(j, carry):
        phase(2 * j, wid + (2 * j) * NW, 0)
        phase(2 * j + 1, wid + (2 * j + 1) * NW, 1)
        return carry

    lax.fori_loop(0, cnt // 2, pair_body, 0)

    @pl.when(cnt % 2 == 1)
    def _():
        phase(cnt - 1, wid + (cnt - 1) * NW, 0)

    for s in (0, 1):
        @pl.when(cnt % 2 == s)
        def _(s=s):
            (cxv, cyv, czv), _, _, csem, _, _ = sets[s]
            pltpu.make_async_copy(cx_hbm.at[pl.ds(0, H)], cxv, csem).wait()
            pltpu.make_async_copy(cy_hbm.at[pl.ds(0, H)], cyv, csem).wait()
            pltpu.make_async_copy(cz_hbm.at[pl.ds(0, H)], czv, csem).wait()

    for s in (0, 1):
        _, _, dsts, _, _, ssem = sets[s]
        for q in range(4):
            pltpu.make_async_copy(
                dsts[q], g2_hbm.at[pl.ds(q * QB, QB), pl.ds(0, C)],
                ssem).wait()


def _fuse_body(lf_ref, g2_ref, out_ref):
    lft = lf_ref[...]
    g2t = g2_ref[...].T
    out_ref[:, 0:H] = lft[:, 0:H] + g2t[0:C, :]
    out_ref[:, H:TB] = lft[:, H:TB] + g2t[C:2 * C, :]


@jax.jit
def _fusion(cx, cy, cz, lft, gf):
    mesh = plsc.VectorSubcoreMesh(core_axis_name="c", subcore_axis_name="s")
    g2 = pl.kernel(
        _gather_body,
        out_type=jax.ShapeDtypeStruct((NB * H, 2 * C), jnp.float32),
        mesh=mesh,
        scratch_types=(
            [pltpu.VMEM((H,), jnp.int32)] * 6
            + [pltpu.VMEM((QB,), jnp.int32)] * 8
            + [pltpu.VMEM((QB, C), jnp.float32)] * 8
            + [pltpu.SemaphoreType.DMA] * 6
        ),
        compiler_params=pltpu.CompilerParams(
            use_tc_tiling_on_sc=False, needs_layout_passes=False),
    )(cx, cy, cz, gf)

    out_t = pl.pallas_call(
        _fuse_body,
        grid=(NB,),
        in_specs=[
            pl.BlockSpec((C, TB), lambda i: (0, i)),
            pl.BlockSpec((H, 2 * C), lambda i: (i, 0)),
        ],
        out_specs=pl.BlockSpec((C, TB), lambda i: (0, i)),
        out_shape=jax.ShapeDtypeStruct((C, N), jnp.float32),
    )(lft, g2)
    return out_t


def kernel(local_features, local_coords, local_base, global_features, global_base):
    adj = (local_coords.astype(jnp.int32)
           + local_base.astype(jnp.int32)[None, :]
           - SCALE * global_base.astype(jnp.int32)[None, :])
    pad = (0, NP - N)
    cx = jnp.pad(adj[:, 0], pad)
    cy = jnp.pad(adj[:, 1], pad)
    cz = jnp.pad(adj[:, 2], pad)
    out_t = _fusion(cx, cy, cz, local_features.T, global_features)
    return out_t.T


# speedup vs baseline: 2.0340x; 1.2451x over previous
import functools

import jax
import jax.numpy as jnp
from jax import lax
from jax.experimental import pallas as pl
from jax.experimental.pallas import tpu as pltpu
from jax.experimental.pallas import tpu_sc as plsc

N = 200000
C = 64
SCALE = 4
GLOBAL_SIZE = 64
N_GLOBAL = GLOBAL_SIZE ** 3

NC = 2
NS = 16
NW = NC * NS

QB = 128
TB = 1024
H = TB // 2
NB = (N + TB - 1) // TB
NP = NB * TB
G = 2 * NB


def _gather_body(cx_hbm, cy_hbm, cz_hbm, gf_hbm, g2_hbm,
                 cx0, cy0, cz0, cx1, cy1, cz1,
                 i00, i01, i02, i03, i10, i11, i12, i13,
                 d00, d01, d02, d03, d10, d11, d12, d13,
                 csem0, csem1, gsem0, gsem1, ssem0, ssem1):
    wid = lax.axis_index("s") * NC + lax.axis_index("c")
    cnt = (G + NW - 1 - wid) // NW

    sets = (
        ((cx0, cy0, cz0), (i00, i01, i02, i03), (d00, d01, d02, d03),
         csem0, gsem0, ssem0),
        ((cx1, cy1, cz1), (i10, i11, i12, i13), (d10, d11, d12, d13),
         csem1, gsem1, ssem1),
    )

    def coords_of(c):
        return (c >> 1) * TB + (c & 1) * H

    def start_coords(c, s):
        (cxv, cyv, czv), _, _, csem, _, _ = sets[s]
        base = jnp.minimum(coords_of(c), NP - H)
        pltpu.make_async_copy(cx_hbm.at[pl.ds(base, H)], cxv, csem).start()
        pltpu.make_async_copy(cy_hbm.at[pl.ds(base, H)], cyv, csem).start()
        pltpu.make_async_copy(cz_hbm.at[pl.ds(base, H)], czv, csem).start()

    def phase(j, c, s):
        (cxv, cyv, czv), idxs, dsts, csem, gsem, ssem = sets[s]
        base = coords_of(c)
        half = c & 1
        base2 = (c >> 1) * H

        pltpu.make_async_copy(cx_hbm.at[pl.ds(base, H)], cxv, csem).wait()
        pltpu.make_async_copy(cy_hbm.at[pl.ds(base, H)], cyv, csem).wait()
        pltpu.make_async_copy(cz_hbm.at[pl.ds(base, H)], czv, csem).wait()

        for t in range(H // 16):
            sl = pl.ds(t * 16, 16)
            x = jnp.clip(cxv[sl] >> 2, 0, GLOBAL_SIZE - 1)
            y = jnp.clip(cyv[sl] >> 2, 0, GLOBAL_SIZE - 1)
            z = jnp.clip(czv[sl] >> 2, 0, GLOBAL_SIZE - 1)
            flat = (x * (GLOBAL_SIZE * GLOBAL_SIZE) + y * GLOBAL_SIZE) + z
            idxs[t // 8][pl.ds((t % 8) * 16, 16)] = flat

        gcps = [pltpu.async_copy(gf_hbm.at[idxs[q]], dsts[q], gsem)
                for q in range(4)]

        start_coords(c + NW, 1 - s)

        @pl.when(j > 1)
        def _():
            for q in range(4):
                pltpu.make_async_copy(
                    dsts[q],
                    g2_hbm.at[pl.ds(base2 + q * QB, QB),
                              pl.ds(half * C, C)],
                    ssem).wait()

        for cp in gcps:
            cp.wait()

        for q in range(4):
            pltpu.make_async_copy(
                dsts[q],
                g2_hbm.at[pl.ds(base2 + q * QB, QB), pl.ds(half * C, C)],
                ssem).start()

    start_coords(wid, 0)

    def pair_body(j, carry):
        phase(2 * j, wid + (2 * j) * NW, 0)
        phase(2 * j + 1, wid + (2 * j + 1) * NW, 1)
        return carry

    lax.fori_loop(0, cnt // 2, pair_body, 0)

    @pl.when(cnt % 2 == 1)
    def _():
        phase(cnt - 1, wid + (cnt - 1) * NW, 0)

    for s in (0, 1):
        @pl.when(cnt % 2 == s)
        def _(s=s):
            (cxv, cyv, czv), _, _, csem, _, _ = sets[s]
            pltpu.make_async_copy(cx_hbm.at[pl.ds(0, H)], cxv, csem).wait()
            pltpu.make_async_copy(cy_hbm.at[pl.ds(0, H)], cyv, csem).wait()
            pltpu.make_async_copy(cz_hbm.at[pl.ds(0, H)], czv, csem).wait()

    for s in (0, 1):
        _, _, dsts, _, _, ssem = sets[s]
        for q in range(4):
            pltpu.make_async_copy(
                dsts[q], g2_hbm.at[pl.ds(q * QB, QB), pl.ds(0, C)],
                ssem).wait()


FB = 4
FW = FB * TB


def _fuse_body(lf_ref, g2_ref, out_ref):
    for b in range(FB):
        lft = lf_ref[:, pl.ds(b * TB, TB)]
        g2t = g2_ref[pl.ds(b * H, H), :].T
        out_ref[:, pl.ds(b * TB, H)] = lft[:, 0:H] + g2t[0:C, :]
        out_ref[:, pl.ds(b * TB + H, H)] = lft[:, H:TB] + g2t[C:2 * C, :]


@jax.jit
def _fusion(cx, cy, cz, lft, gf):
    mesh = plsc.VectorSubcoreMesh(core_axis_name="c", subcore_axis_name="s")
    g2 = pl.kernel(
        _gather_body,
        out_type=jax.ShapeDtypeStruct((NB * H, 2 * C), jnp.float32),
        mesh=mesh,
        scratch_types=(
            [pltpu.VMEM((H,), jnp.int32)] * 6
            + [pltpu.VMEM((QB,), jnp.int32)] * 8
            + [pltpu.VMEM((QB, C), jnp.float32)] * 8
            + [pltpu.SemaphoreType.DMA] * 6
        ),
        compiler_params=pltpu.CompilerParams(
            use_tc_tiling_on_sc=False, needs_layout_passes=False),
    )(cx, cy, cz, gf)

    out_t = pl.pallas_call(
        _fuse_body,
        grid=(NB // FB,),
        in_specs=[
            pl.BlockSpec((C, FW), lambda i: (0, i)),
            pl.BlockSpec((FB * H, 2 * C), lambda i: (i, 0)),
        ],
        out_specs=pl.BlockSpec((C, FW), lambda i: (0, i)),
        out_shape=jax.ShapeDtypeStruct((C, N), jnp.float32),
    )(lft, g2)
    return out_t


def kernel(local_features, local_coords, local_base, global_features, global_base):
    adj = (local_coords.astype(jnp.int32)
           + local_base.astype(jnp.int32)[None, :]
           - SCALE * global_base.astype(jnp.int32)[None, :])
    pad = (0, NP - N)
    cx = jnp.pad(adj[:, 0], pad)
    cy = jnp.pad(adj[:, 1], pad)
    cz = jnp.pad(adj[:, 2], pad)
    out_t = _fusion(cx, cy, cz, local_features.T, global_features)
    return out_t.T
